# Initial kernel scaffold; baseline (speedup 1.0000x reference)
import jax
import jax.numpy as jnp
from jax.experimental import pallas as pl


def kernel(A, lengths):
    raise NotImplementedError("write your pallas kernel here")


# speedup vs baseline: 1.1603x; 1.1603x over previous
import jax
import jax.numpy as jnp
from jax.experimental import pallas as pl
from jax.experimental.pallas import tpu as pltpu

_N = 512
_B = 16

_ROT = ((13, 15, 26, 6), (17, 29, 16, 24))
_KS = (0x0, 0x2A, 0x1BD11BDA ^ 0x0 ^ 0x2A)


def _threefry_bits(ctr):
    x0 = jnp.zeros_like(ctr) + jnp.uint32(_KS[0])
    x1 = ctr + jnp.uint32(_KS[1])
    for i in range(5):
        for r in _ROT[i % 2]:
            x0 = x0 + x1
            x1 = (x1 << jnp.uint32(r)) | (x1 >> jnp.uint32(32 - r))
            x1 = x1 ^ x0
        x0 = x0 + jnp.uint32(_KS[(i + 1) % 3])
        x1 = x1 + jnp.uint32(_KS[(i + 2) % 3] + i + 1)
    return x0 ^ x1


def _bits_to_uniform(bits):
    fb = (bits >> jnp.uint32(9)) | jnp.uint32(0x3F800000)
    f = jax.lax.bitcast_convert_type(fb, jnp.float32) - jnp.float32(1.0)
    minv = jnp.float32(1e-6)
    span = jnp.float32((1.0 - 1e-6) - 1e-6)
    return jnp.maximum(minv, f * span + minv)


def _body(len_ref, a_ref, o_ref):
    b = pl.program_id(0)
    a = a_ref[0]
    rows = jax.lax.broadcasted_iota(jnp.int32, (_N, _N), 0)
    cols = jax.lax.broadcasted_iota(jnp.int32, (_N, _N), 1)
    ctr = (b * (_N * _N) + rows * _N + cols).astype(jnp.uint32)
    u = _bits_to_uniform(_threefry_bits(ctr))
    y = u / (u + (jnp.float32(1.0) - u) * jnp.exp(-a))
    ln = len_ref[b]
    m = (rows < ln) & (cols < ln) & (rows != cols)
    o_ref[0] = jnp.where(m, y, jnp.float32(0.0))


def kernel(A, lengths):
    lengths32 = lengths.astype(jnp.int32)
    return pl.pallas_call(
        _body,
        grid=(_B,),
        in_specs=[
            pl.BlockSpec(memory_space=pltpu.SMEM),
            pl.BlockSpec((1, _N, _N), lambda b: (b, 0, 0)),
        ],
        out_specs=pl.BlockSpec((1, _N, _N), lambda b: (b, 0, 0)),
        out_shape=jax.ShapeDtypeStruct((_B, _N, _N), jnp.float32),
    )(lengths32, A)
